# baseline (speedup 1.0000x reference)
import functools

import jax
import jax.numpy as jnp
from jax import lax
from jax.experimental import pallas as pl
from jax.experimental.pallas import tpu as pltpu
from jax.experimental.pallas import tpu_sc as plsc

VOCAB_SIZE = 1000000
EMBED_DIM = 32
BATCH = 16384
N_FIELDS = 26

NUM_CORES = 2
NUM_SUBCORES = 16
NUM_WORKERS = NUM_CORES * NUM_SUBCORES

TOTAL_ROWS = BATCH * N_FIELDS
ROWS_PER_WORKER = TOTAL_ROWS // NUM_WORKERS

N_VTILES = VOCAB_SIZE // 128
TAIL_V0 = VOCAB_SIZE - 128
BASE_T, EXTRA_T = divmod(N_VTILES, NUM_WORKERS)

CHUNK = 512
N_CHUNKS = ROWS_PER_WORKER // CHUNK
assert N_CHUNKS * CHUNK == ROWS_PER_WORKER
assert BATCH % CHUNK == 0

_mesh = plsc.VectorSubcoreMesh(
    core_axis_name="c", subcore_axis_name="s",
    num_cores=NUM_CORES, num_subcores=NUM_SUBCORES,
)


@functools.partial(
    pl.kernel,
    mesh=_mesh,
    compiler_params=pltpu.CompilerParams(needs_layout_passes=False),
    out_type=jax.ShapeDtypeStruct((VOCAB_SIZE * EMBED_DIM // 128, 128),
                                  jnp.float32),
    scratch_types=[
        pltpu.VMEM((8, 512), jnp.float32),
        pltpu.VMEM((8, 512), jnp.float32),
        pltpu.VMEM((EMBED_DIM, 128), jnp.float32),
        pltpu.VMEM((EMBED_DIM, 128), jnp.float32),
        pltpu.SemaphoreType.DMA,
        pltpu.SemaphoreType.DMA,
        pltpu.SemaphoreType.DMA,
        pltpu.SemaphoreType.DMA,
    ],
)
def _transpose_kernel(tableT_hbm, tail_hbm, out_hbm,
                      src0, src1, dst0, dst1, gs0, gs1, os0, os1):
    wid = lax.axis_index("s") * NUM_CORES + lax.axis_index("c")
    nt = jnp.where(wid < EXTRA_T, BASE_T + 1, BASE_T)
    t0 = wid * BASE_T + jnp.minimum(wid, EXTRA_T)

    iota16 = lax.iota(jnp.int32, 16)
    rvec = lax.rem(iota16, 8)
    cv = (lax.div(iota16, 8)) * 128
    srcs = (src0, src1)
    dsts = (dst0, dst1)
    gsems = (gs0, gs1)
    osems = (os0, os1)

    def load_tile(vt, b):
        copies = []
        for eb in range(4):
            copies.append(pltpu.async_copy(
                tableT_hbm.at[pl.ds(eb * 8, 8), pl.ds(vt * 128, 128)],
                srcs[b].at[:, pl.ds(eb * 128, 128)], gsems[b]))
        return copies

    def load_tail(b):
        copies = []
        for eb in range(4):
            copies.append(pltpu.async_copy(
                tail_hbm.at[pl.ds(eb * 8, 8)],
                srcs[b].at[:, pl.ds(eb * 128, 128)], gsems[b]))
        return copies

    def transpose_block(src, dst):
        for qL in range(32):
            for j0 in range(0, 128, 16):
                e0 = j0 % 32
                vL = 4 * qL + j0 // 32
                cvec = cv + (e0 // 8) * 128 + vL
                dst[qL, pl.ds(j0, 16)] = plsc.load_gather(src, [rvec, cvec])

    def tile_pair(p, carry):
        for b in range(2):
            vt = t0 + p * 2 + b

            @pl.when(vt < t0 + nt)
            def _():
                for c in load_tile(vt, b):
                    c.wait()
                transpose_block(srcs[b], dsts[b])
                pltpu.async_copy(
                    dsts[b], out_hbm.at[pl.ds(vt * 32, 32)], osems[b]).wait()
        return carry

    lax.fori_loop(0, (BASE_T + 2) // 2, tile_pair, 0)

    @pl.when(wid == NUM_WORKERS - 1)
    def _():
        for c in load_tail(0):
            c.wait()
        transpose_block(src0, dst0)
        pltpu.async_copy(
            dst0, out_hbm.at[pl.ds(TAIL_V0 // 4, 32)], os0).wait()


@functools.partial(
    pl.kernel,
    mesh=_mesh,
    compiler_params=pltpu.CompilerParams(use_tc_tiling_on_sc=False,
                                         needs_layout_passes=False),
    out_type=jax.ShapeDtypeStruct((N_FIELDS, EMBED_DIM, BATCH), jnp.float32),
    scratch_types=[
        pltpu.VMEM((ROWS_PER_WORKER,), jnp.int32),
        pltpu.VMEM((CHUNK, EMBED_DIM), jnp.float32),
        pltpu.VMEM((CHUNK, EMBED_DIM), jnp.float32),
        pltpu.VMEM((EMBED_DIM, CHUNK), jnp.float32),
        pltpu.VMEM((EMBED_DIM, CHUNK), jnp.float32),
        pltpu.SemaphoreType.DMA,
        pltpu.SemaphoreType.DMA,
        pltpu.SemaphoreType.DMA,
        pltpu.SemaphoreType.DMA,
    ],
)
def _gather_kernel(idx_hbm, table_hbm, out_hbm, idx_v,
                   rows0, rows1, tr0, tr1, gs0, gs1, os0, os1):
    wid = lax.axis_index("s") * NUM_CORES + lax.axis_index("c")
    base = wid * ROWS_PER_WORKER

    rows = (rows0, rows1)
    trs = (tr0, tr1)
    gsems = (gs0, gs1)
    osems = (os0, os1)

    iota16 = lax.iota(jnp.int32, 16)

    pltpu.sync_copy(idx_hbm.at[pl.ds(base, ROWS_PER_WORKER)], idx_v)

    def gather(g):
        b = g & 1
        return pltpu.async_copy(
            table_hbm.at[idx_v.at[pl.ds(g * CHUNK, CHUNK)]], rows[b], gsems[b])

    def transpose_chunk(src, dst):
        def e_body(e, carry):
            eb = jnp.broadcast_to(e, (16,))
            for b16 in range(CHUNK // 16):
                dst[e, pl.ds(b16 * 16, 16)] = plsc.load_gather(
                    src, [iota16 + b16 * 16, eb])
            return carry
        lax.fori_loop(0, EMBED_DIM, e_body, 0)

    def writeback(g, b):
        j0 = base + g * CHUNK
        f = j0 // BATCH
        b0 = j0 % BATCH
        return pltpu.async_copy(
            trs[b], out_hbm.at[f, :, pl.ds(b0, CHUNK)], osems[b])

    gathers = [None] * N_CHUNKS
    writebacks = [None] * N_CHUNKS
    gathers[0] = gather(0)
    for g in range(N_CHUNKS):
        b = g & 1
        if g + 1 < N_CHUNKS:
            gathers[g + 1] = gather(g + 1)
        gathers[g].wait()
        if g >= 2:
            writebacks[g - 2].wait()
        transpose_chunk(rows[b], trs[b])
        writebacks[g] = writeback(g, b)
    writebacks[N_CHUNKS - 2].wait()
    writebacks[N_CHUNKS - 1].wait()


def kernel(ability_name, ability_embed_weight):
    tableT = ability_embed_weight.T
    tail = lax.slice(tableT, (0, TAIL_V0), (EMBED_DIM, VOCAB_SIZE))
    table_rm = _transpose_kernel(tableT, tail)
    table_rows = table_rm.reshape(VOCAB_SIZE, EMBED_DIM)
    flat_idx = ability_name.T.reshape(TOTAL_ROWS)
    outT = _gather_kernel(flat_idx, table_rows)
    return outT.transpose(2, 0, 1)


# speedup vs baseline: 1.8227x; 1.8227x over previous
import functools

import jax
import jax.numpy as jnp
from jax import lax
from jax.experimental import pallas as pl
from jax.experimental.pallas import tpu as pltpu
from jax.experimental.pallas import tpu_sc as plsc

VOCAB_SIZE = 1000000
EMBED_DIM = 32
BATCH = 16384
N_FIELDS = 26

NUM_CORES = 2
NUM_SUBCORES = 16
NUM_WORKERS = NUM_CORES * NUM_SUBCORES

TOTAL_ROWS = BATCH * N_FIELDS
ROWS_PER_WORKER = TOTAL_ROWS // NUM_WORKERS
CHUNK = 1664
N_CHUNKS = ROWS_PER_WORKER // CHUNK

assert ROWS_PER_WORKER * NUM_WORKERS == TOTAL_ROWS
assert N_CHUNKS * CHUNK == ROWS_PER_WORKER and N_CHUNKS >= 2

_mesh = plsc.VectorSubcoreMesh(
    core_axis_name="c", subcore_axis_name="s",
    num_cores=NUM_CORES, num_subcores=NUM_SUBCORES,
)


@functools.partial(
    pl.kernel,
    mesh=_mesh,
    compiler_params=pltpu.CompilerParams(use_tc_tiling_on_sc=False),
    out_type=jax.ShapeDtypeStruct((TOTAL_ROWS, EMBED_DIM), jnp.float32),
    scratch_types=[
        pltpu.VMEM((ROWS_PER_WORKER,), jnp.int32),
        pltpu.VMEM((CHUNK, EMBED_DIM), jnp.float32),
        pltpu.VMEM((CHUNK, EMBED_DIM), jnp.float32),
        pltpu.SemaphoreType.DMA,
        pltpu.SemaphoreType.DMA,
        pltpu.SemaphoreType.DMA,
        pltpu.SemaphoreType.DMA,
    ],
)
def _gather_kernel(idx_hbm, table_hbm, out_hbm, idx_v, rows0, rows1,
                   gsem0, gsem1, osem0, osem1):
    wid = lax.axis_index("s") * NUM_CORES + lax.axis_index("c")
    base = wid * ROWS_PER_WORKER

    rows = (rows0, rows1)
    gsems = (gsem0, gsem1)
    osems = (osem0, osem1)

    pltpu.sync_copy(idx_hbm.at[pl.ds(base, ROWS_PER_WORKER)], idx_v)

    def gather(g):
        b = g & 1
        return pltpu.async_copy(
            table_hbm.at[idx_v.at[pl.ds(g * CHUNK, CHUNK)]], rows[b], gsems[b])

    gathers = [None] * N_CHUNKS
    writebacks = [None] * N_CHUNKS
    gathers[0] = gather(0)
    for g in range(N_CHUNKS):
        b = g & 1
        if g + 1 < N_CHUNKS:
            if g >= 1:
                writebacks[g - 1].wait()
            gathers[g + 1] = gather(g + 1)
        gathers[g].wait()
        writebacks[g] = pltpu.async_copy(
            rows[b], out_hbm.at[pl.ds(base + g * CHUNK, CHUNK)], osems[b])
    writebacks[N_CHUNKS - 2].wait()
    writebacks[N_CHUNKS - 1].wait()


def kernel(ability_name, ability_embed_weight):
    flat_idx = ability_name.reshape(TOTAL_ROWS)
    out = _gather_kernel(flat_idx, ability_embed_weight)
    return out.reshape(BATCH, N_FIELDS, EMBED_DIM)
